# Initial kernel scaffold; baseline (speedup 1.0000x reference)
import jax
import jax.numpy as jnp
from jax.experimental import pallas as pl


def kernel(x, table):
    raise NotImplementedError("write your pallas kernel here")


# speedup vs baseline: 1.6822x; 1.6822x over previous
import functools

import jax
import jax.numpy as jnp
from jax import lax
from jax.experimental import pallas as pl
from jax.experimental.pallas import tpu as pltpu
from jax.experimental.pallas import tpu_sc as plsc

D_MODEL = 64
NC, NS = 2, 16
NW = NC * NS
K = 128


@functools.lru_cache(maxsize=None)
def _make_sc_gather(n_total: int, vocab: int):
    per_w = n_total // NW
    n_chunks = per_w // K
    mesh = plsc.VectorSubcoreMesh(core_axis_name="c", subcore_axis_name="s")

    @functools.partial(
        pl.kernel,
        out_type=jax.ShapeDtypeStruct((n_total, D_MODEL), jnp.float32),
        mesh=mesh,
        scratch_types=[
            pltpu.VMEM((n_chunks, K), jnp.int32),
            pltpu.VMEM((K, D_MODEL), jnp.float32),
            pltpu.SemaphoreType.DMA,
        ],
        compiler_params=pltpu.CompilerParams(use_tc_tiling_on_sc=False),
    )
    def gather_kernel(x_hbm, table_hbm, out_hbm, idx_v, rows_v, gsem):
        wid = lax.axis_index("s") * NC + lax.axis_index("c")
        base_chunk = wid * n_chunks
        pltpu.sync_copy(x_hbm.at[pl.ds(base_chunk, n_chunks)], idx_v)

        def chunk(j, carry):
            pltpu.async_copy(table_hbm.at[idx_v.at[j]], rows_v, gsem).wait()
            pltpu.sync_copy(rows_v, out_hbm.at[pl.ds((base_chunk + j) * K, K)])
            return carry

        lax.fori_loop(0, n_chunks, chunk, 0)

    return gather_kernel


def kernel(x, table):
    b, h = x.shape
    n = b * h
    xf = x.reshape(n // K, K).astype(jnp.int32)
    out = _make_sc_gather(n, table.shape[0])(xf, table)
    return out.reshape(b, h, D_MODEL)


# speedup vs baseline: 1.8755x; 1.1149x over previous
import functools

import jax
import jax.numpy as jnp
from jax import lax
from jax.experimental import pallas as pl
from jax.experimental.pallas import tpu as pltpu
from jax.experimental.pallas import tpu_sc as plsc

D_MODEL = 64
NC, NS = 2, 16
NW = NC * NS
K = 128
NBUF = 5
GROUP_ROWS = NBUF * K


@functools.lru_cache(maxsize=None)
def _make_sc_gather(n_total: int):
    per_w = n_total // NW
    n_chunks = per_w // K
    n_groups = n_chunks // NBUF
    n_pairs = n_groups // 2
    assert n_total == NW * n_chunks * K and n_groups == 2 * n_pairs
    mesh = plsc.VectorSubcoreMesh(core_axis_name="c", subcore_axis_name="s")

    @functools.partial(
        pl.kernel,
        out_type=jax.ShapeDtypeStruct((n_total, D_MODEL), jnp.float32),
        mesh=mesh,
        scratch_types=[
            pltpu.VMEM((n_chunks, K), jnp.int32),
            pltpu.VMEM((2, GROUP_ROWS, D_MODEL), jnp.float32),
            pltpu.SemaphoreType.DMA,
            pltpu.SemaphoreType.DMA,
            pltpu.SemaphoreType.DMA,
            pltpu.SemaphoreType.DMA,
        ],
        compiler_params=pltpu.CompilerParams(use_tc_tiling_on_sc=False),
    )
    def gather_kernel(x_hbm, table_hbm, out_hbm, idx_v, rows_v, gs0, gs1,
                      os0, os1):
        wid = lax.axis_index("s") * NC + lax.axis_index("c")
        base_chunk = wid * n_chunks
        base_row = base_chunk * K
        pltpu.sync_copy(x_hbm.at[pl.ds(base_chunk, n_chunks)], idx_v)

        def fire_gathers(g, slot, gsem):
            for b in range(NBUF):
                pltpu.async_copy(
                    table_hbm.at[idx_v.at[g * NBUF + b]],
                    rows_v.at[slot, pl.ds(b * K, K)],
                    gsem)

        def wait_group(slot, sem):
            pltpu.make_async_copy(
                out_hbm.at[pl.ds(0, GROUP_ROWS)], rows_v.at[slot], sem).wait()

        def fire_out(g, slot, osem):
            pltpu.async_copy(
                rows_v.at[slot],
                out_hbm.at[pl.ds(base_row + g * GROUP_ROWS, GROUP_ROWS)],
                osem)

        fire_gathers(0, 0, gs0)

        def pair(k, carry):
            a = 2 * k

            @pl.when(k > 0)
            def _():
                wait_group(1, os1)

            fire_gathers(a + 1, 1, gs1)
            wait_group(0, gs0)
            fire_out(a, 0, os0)
            wait_group(0, os0)

            @pl.when(k < n_pairs - 1)
            def _():
                fire_gathers(a + 2, 0, gs0)

            wait_group(1, gs1)
            fire_out(a + 1, 1, os1)
            return carry

        lax.fori_loop(0, n_pairs, pair, 0)
        wait_group(1, os1)

    return gather_kernel


def kernel(x, table):
    b, h = x.shape
    n = b * h
    xf = x.reshape(n // K, K).astype(jnp.int32)
    out = _make_sc_gather(n)(xf, table)
    return out.reshape(b, h, D_MODEL)
